# Initial kernel scaffold; baseline (speedup 1.0000x reference)
import jax
import jax.numpy as jnp
from jax.experimental import pallas as pl


def kernel(item_features, w0, w1, k_param, graph_indices, graph_values, k, b):
    raise NotImplementedError("write your pallas kernel here")


# speedup vs baseline: 10.5252x; 10.5252x over previous
import functools

import jax
import jax.numpy as jnp
from jax.experimental import pallas as pl
from jax.experimental.pallas import tpu as pltpu

M_ITEMS = 4000
D_FEAT = 256
K_TOP = 20
N_USERS = 6000
B_ROWS = 400


def _emb_kernel(x_ref, w0_ref, w1_ref, out_ref):
    x = x_ref[...]
    h = jnp.maximum(x * w0_ref[...], 0.0) * w1_ref[...]
    norm = jnp.sqrt(jnp.sum(h * h, axis=1, keepdims=True))
    out_ref[...] = h / (norm + 1e-8)


def _topk_kernel(emb_blk_ref, emb_all_ref, vals_ref, idx_ref):
    a = emb_blk_ref[...]
    bm = emb_all_ref[...]
    sim = jax.lax.dot_general(a, bm, (((1,), (1,)), ((), ())),
                              preferred_element_type=jnp.float32)
    col = jax.lax.broadcasted_iota(jnp.int32, sim.shape, 1)
    big = jnp.int32(2 ** 30)
    neg = jnp.float32(-jnp.inf)
    vs, ids = [], []
    for _ in range(K_TOP):
        m = jnp.max(sim, axis=1, keepdims=True)
        amin = jnp.min(jnp.where(sim == m, col, big), axis=1, keepdims=True)
        vs.append(m)
        ids.append(amin)
        sim = jnp.where(col == amin, neg, sim)
    vals_ref[...] = jnp.concatenate(vs, axis=1)
    idx_ref[...] = jnp.concatenate(ids, axis=1)


def kernel(item_features, w0, w1, k_param, graph_indices, graph_values, k, b):
    emb = pl.pallas_call(
        _emb_kernel,
        out_shape=jax.ShapeDtypeStruct((M_ITEMS, D_FEAT), jnp.float32),
    )(item_features, w0.reshape(1, D_FEAT), w1.reshape(1, D_FEAT))

    grid = (M_ITEMS // B_ROWS,)
    vals, idx = pl.pallas_call(
        _topk_kernel,
        grid=grid,
        in_specs=[
            pl.BlockSpec((B_ROWS, D_FEAT), lambda i: (i, 0)),
            pl.BlockSpec((M_ITEMS, D_FEAT), lambda i: (0, 0)),
        ],
        out_specs=[
            pl.BlockSpec((B_ROWS, K_TOP), lambda i: (i, 0)),
            pl.BlockSpec((B_ROWS, K_TOP), lambda i: (i, 0)),
        ],
        out_shape=[
            jax.ShapeDtypeStruct((M_ITEMS, K_TOP), jnp.float32),
            jax.ShapeDtypeStruct((M_ITEMS, K_TOP), jnp.int32),
        ],
    )(emb, emb)

    rows = jnp.repeat(jnp.arange(M_ITEMS, dtype=jnp.int32), K_TOP) + N_USERS
    cols = idx.reshape(-1) + N_USERS
    w = jnp.where(vals >= b, vals, jnp.zeros_like(vals)).reshape(-1)
    row_cat = jnp.concatenate([rows, cols])
    col_cat = jnp.concatenate([cols, rows])
    new_indices = jnp.stack([row_cat, col_cat], axis=0)
    out_indices = jnp.concatenate([graph_indices, new_indices], axis=1)
    out_values = jnp.concatenate([jnp.ones_like(graph_values), w, w])
    return out_indices, out_values
